# Initial kernel scaffold; baseline (speedup 1.0000x reference)
import jax
import jax.numpy as jnp
from jax.experimental import pallas as pl


def kernel(x, batch, params):
    raise NotImplementedError("write your pallas kernel here")


# speedup vs baseline: 4.8229x; 4.8229x over previous
import jax
import jax.numpy as jnp
from jax.experimental import pallas as pl
from jax.experimental.pallas import tpu as pltpu

G = 128
P = 256
K = 16
EPS = 1e-5
HI = jax.lax.Precision.HIGHEST


def _leaky(v):
    return jnp.where(v >= 0, v, 0.2 * v)


def _bn_apply(h, bn):
    return (h - bn[0:1]) / bn[1:2] * bn[2:3] + bn[3:4]


def _edge_conv(h, cnt, row2, col2, rowv, W1T, bn1, W2T, bn2):
    hh = jnp.dot(h, h.T, precision=HI, preferred_element_type=jnp.float32)
    n2 = jnp.sum(jnp.where(col2 == row2, hh, 0.0), axis=1)
    D = n2[:, None] + n2[None, :] - 2.0 * hh
    D = jnp.where(col2 == row2, 0.0, D)
    D = jnp.where(col2 >= cnt, jnp.inf, D)
    row_ok = rowv < cnt
    A = jnp.zeros((P, P), dtype=jnp.bool_)
    for k in range(K + 1):
        m = jnp.min(D, axis=1)
        j = jnp.min(jnp.where(D <= m[:, None], col2, P), axis=1)
        oh = col2 == j[:, None]
        D = jnp.where(oh, jnp.inf, D)
        if k > 0:
            A = A | (oh & row_ok & (k < cnt))
    AT = A.T.astype(jnp.int32)

    def cond(carry):
        _, _, rem = carry
        return rem > 0

    def body(carry):
        at, out, _ = carry
        atb = at > 0
        qidx = jnp.min(jnp.where(atb, col2, P), axis=1)
        colnz = (qidx < P)[:, None]
        osel = atb & (col2 == qidx[:, None])
        hq = jnp.dot(osel.astype(jnp.float32), h, precision=HI,
                     preferred_element_type=jnp.float32)
        m = jnp.concatenate([h, hq - h], axis=1)
        u = _leaky(_bn_apply(
            jnp.dot(m, W1T, preferred_element_type=jnp.float32), bn1))
        e = _leaky(_bn_apply(
            jnp.dot(u, W2T, preferred_element_type=jnp.float32), bn2))
        out = jnp.where(colnz, jnp.maximum(out, e), out)
        at = jnp.where(osel, 0, at)
        return at, out, jnp.sum(at)

    init = jnp.full((P, W2T.shape[1]), -jnp.inf, dtype=jnp.float32)
    _, acc, _ = jax.lax.while_loop(cond, body, (AT, init, jnp.sum(AT)))
    out = jnp.where(jnp.isneginf(acc), 0.0, acc)
    return jnp.where(row_ok, out, 0.0)


def _graph_kernel(counts_ref, xp_ref, Wp_ref, bnp_ref,
                  W11_ref, bn11_ref, W21_ref, bn21_ref,
                  W12_ref, bn12_ref, W22_ref, bn22_ref,
                  W13_ref, bn13_ref, W23_ref, bn23_ref,
                  out_ref):
    g = pl.program_id(0)
    cnt = counts_ref[0, g]
    row2 = jax.lax.broadcasted_iota(jnp.int32, (P, P), 0)
    col2 = jax.lax.broadcasted_iota(jnp.int32, (P, P), 1)
    rowv = jax.lax.broadcasted_iota(jnp.int32, (P, 1), 0)

    x = xp_ref[0]
    h = _leaky(_bn_apply(
        jnp.dot(x, Wp_ref[...], preferred_element_type=jnp.float32),
        bnp_ref[...]))
    h = jnp.where(rowv < cnt, h, 0.0)

    h1 = _edge_conv(h, cnt, row2, col2, rowv,
                    W11_ref[...], bn11_ref[...], W21_ref[...], bn21_ref[...])
    h2 = _edge_conv(h1, cnt, row2, col2, rowv,
                    W12_ref[...], bn12_ref[...], W22_ref[...], bn22_ref[...])
    h3 = _edge_conv(h2, cnt, row2, col2, rowv,
                    W13_ref[...], bn13_ref[...], W23_ref[...], bn23_ref[...])

    cntf = jnp.maximum(cnt.astype(jnp.float32), 1.0)
    off = 0
    for hpart in (h1, h2, h3):
        cpart = hpart.shape[1]
        mean = jnp.sum(hpart, axis=0) / cntf
        mx = jnp.max(jnp.where(rowv < cnt, hpart, -jnp.inf), axis=0)
        mx = jnp.where(jnp.isneginf(mx), 0.0, mx)
        out_ref[0, 0, off:off + cpart] = mean
        out_ref[0, 0, 448 + off:448 + off + cpart] = mx
        off += cpart


def _head_kernel(pooled_ref, W1_ref, bn1_ref, W2_ref, bn2_ref,
                 Wo_ref, bo_ref, out_ref):
    o = _leaky(_bn_apply(
        jnp.dot(pooled_ref[...], W1_ref[...],
                preferred_element_type=jnp.float32), bn1_ref[...]))
    o = _leaky(_bn_apply(
        jnp.dot(o, W2_ref[...], preferred_element_type=jnp.float32),
        bn2_ref[...]))
    out_ref[...] = jnp.dot(o, Wo_ref[...],
                           preferred_element_type=jnp.float32) + bo_ref[...]


def _bn_pack(bn):
    return jnp.stack([bn["rm"], jnp.sqrt(bn["rv"] + EPS), bn["g"], bn["b"]]
                     ).astype(jnp.float32)


def kernel(x, batch, params):
    n = x.shape[0]
    x = x.astype(jnp.float32)
    b32 = batch.astype(jnp.int32)
    counts = jnp.bincount(b32, length=G).astype(jnp.int32)
    starts = jnp.cumsum(counts) - counts
    pos = jnp.arange(n, dtype=jnp.int32) - starts[b32]
    xp = jnp.zeros((G, P, 128), jnp.float32).at[b32, pos, :x.shape[1]].set(x)

    WpT = jnp.zeros((128, 64), jnp.float32).at[:x.shape[1]].set(
        params["proj_W"].T.astype(jnp.float32))
    weights = [WpT, _bn_pack(params["proj_bn"])]
    for name in ("ec1", "ec2", "ec3"):
        p = params[name]
        weights += [p["W1"].T.astype(jnp.float32), _bn_pack(p["bn1"]),
                    p["W2"].T.astype(jnp.float32), _bn_pack(p["bn2"])]

    full = lambda a: pl.BlockSpec(a.shape, lambda g: (0,) * a.ndim)
    pooled = pl.pallas_call(
        _graph_kernel,
        grid=(G,),
        in_specs=[
            pl.BlockSpec(memory_space=pltpu.SMEM),
            pl.BlockSpec((1, P, 128), lambda g: (g, 0, 0)),
            *[full(w) for w in weights],
        ],
        out_specs=pl.BlockSpec((1, 1, 896), lambda g: (g, 0, 0)),
        out_shape=jax.ShapeDtypeStruct((G, 1, 896), jnp.float32),
        compiler_params=pltpu.CompilerParams(
            dimension_semantics=("arbitrary",)),
    )(counts.reshape(1, G), xp, *weights)
    pooled = pooled.reshape(G, 896)

    Wo = jnp.zeros((256, 128), jnp.float32).at[:, :2].set(
        params["out_W"].T.astype(jnp.float32))
    bo = jnp.zeros((1, 128), jnp.float32).at[0, :2].set(
        params["out_b"].astype(jnp.float32))
    head = [params["fc1_W"].T.astype(jnp.float32), _bn_pack(params["fc1_bn"]),
            params["fc2_W"].T.astype(jnp.float32), _bn_pack(params["fc2_bn"]),
            Wo, bo]
    out = pl.pallas_call(
        _head_kernel,
        in_specs=[pl.BlockSpec(pooled.shape, lambda: (0, 0)),
                  *[pl.BlockSpec(w.shape, lambda: (0, 0)) for w in head]],
        out_specs=pl.BlockSpec((G, 128), lambda: (0, 0)),
        out_shape=jax.ShapeDtypeStruct((G, 128), jnp.float32),
    )(pooled, *head)
    return out[:, :2]
